# baseline (speedup 1.0000x reference)
import functools
import math

import jax
import jax.numpy as jnp
from jax import lax
from jax.experimental import pallas as pl
from jax.experimental.pallas import tpu as pltpu
from jax.experimental.pallas import tpu_sc as plsc

D_MODEL = 1024
SCALE = math.sqrt(D_MODEL)
LANES = 16
VREGS_PER_ROW = D_MODEL // LANES
NUM_CORES = 2
NUM_SUBCORES = 16
NUM_WORKERS = NUM_CORES * NUM_SUBCORES
CHUNK = 16
NBUF = 4


def _sc_body(n_chunks, table_hbm, idx_hbm, out_hbm, idx_v, *refs):
    bufs = refs[:NBUF]
    gsems = refs[NBUF:2 * NBUF]
    ssems = refs[2 * NBUF:3 * NBUF]
    wid = lax.axis_index("s") * NUM_CORES + lax.axis_index("c")
    base = wid * (n_chunks * CHUNK)
    pltpu.sync_copy(idx_hbm.at[wid], idx_v)

    gather = {}
    store = {}
    for j in range(min(NBUF - 1, n_chunks)):
        gather[j % NBUF] = pltpu.async_copy(
            table_hbm.at[idx_v.at[j]], bufs[j % NBUF], gsems[j % NBUF])
    for j in range(n_chunks):
        b = j % NBUF
        nxt = j + NBUF - 1
        if nxt < n_chunks:
            nb = nxt % NBUF
            if nxt >= NBUF:
                store[nb].wait()
            gather[nb] = pltpu.async_copy(
                table_hbm.at[idx_v.at[nxt]], bufs[nb], gsems[nb])
        gather[b].wait()
        cur = bufs[b]

        def scale_row(r, carry):
            for v in range(VREGS_PER_ROW):
                sl = pl.ds(v * LANES, LANES)
                cur[r, sl] = cur[r, sl] * SCALE
            return carry

        lax.fori_loop(0, CHUNK, scale_row, 0, unroll=False)
        store[b] = pltpu.async_copy(
            cur, out_hbm.at[pl.ds(base + j * CHUNK, CHUNK)], ssems[b])
    for j in range(max(0, n_chunks - NBUF), n_chunks):
        store[j % NBUF].wait()


@functools.lru_cache(maxsize=None)
def _make_lookup(batch):
    assert batch % (NUM_WORKERS * CHUNK) == 0
    n_chunks = batch // (NUM_WORKERS * CHUNK)
    mesh = plsc.VectorSubcoreMesh(core_axis_name="c", subcore_axis_name="s")
    return pl.kernel(
        functools.partial(_sc_body, n_chunks),
        out_type=jax.ShapeDtypeStruct((batch, D_MODEL), jnp.float32),
        mesh=mesh,
        scratch_types=(
            [pltpu.VMEM((n_chunks, CHUNK), jnp.int32)]
            + [pltpu.VMEM((CHUNK, D_MODEL), jnp.float32)] * NBUF
            + [pltpu.SemaphoreType.DMA] * (2 * NBUF)
        ),
    )


def kernel(x, table):
    batch = x.size
    n_chunks = batch // (NUM_WORKERS * CHUNK)
    idx = x.astype(jnp.int32).reshape(NUM_WORKERS, n_chunks, CHUNK)
    out = _make_lookup(batch)(table, idx)
    return out.reshape(*x.shape, D_MODEL)


# speedup vs baseline: 1.0492x; 1.0492x over previous
import functools
import math

import jax
import jax.numpy as jnp
from jax import lax
from jax.experimental import pallas as pl
from jax.experimental.pallas import tpu as pltpu
from jax.experimental.pallas import tpu_sc as plsc

D_MODEL = 1024
SCALE = math.sqrt(D_MODEL)
LANES = 16
VREGS_PER_ROW = D_MODEL // LANES
NUM_CORES = 2
NUM_SUBCORES = 16
NUM_WORKERS = NUM_CORES * NUM_SUBCORES
CHUNK = 32
NBUF = 3


def _sc_body(n_chunks, table_hbm, idx_hbm, out_hbm, idx_v, *refs):
    bufs = refs[:NBUF]
    gsems = refs[NBUF:2 * NBUF]
    ssems = refs[2 * NBUF:3 * NBUF]
    wid = lax.axis_index("s") * NUM_CORES + lax.axis_index("c")
    base = wid * (n_chunks * CHUNK)
    pltpu.sync_copy(idx_hbm.at[wid], idx_v)

    gather = {}
    store = {}
    for j in range(min(NBUF - 1, n_chunks)):
        gather[j % NBUF] = pltpu.async_copy(
            table_hbm.at[idx_v.at[j]], bufs[j % NBUF], gsems[j % NBUF])
    for j in range(n_chunks):
        b = j % NBUF
        nxt = j + NBUF - 1
        if nxt < n_chunks:
            nb = nxt % NBUF
            if nxt >= NBUF:
                store[nb].wait()
            gather[nb] = pltpu.async_copy(
                table_hbm.at[idx_v.at[nxt]], bufs[nb], gsems[nb])
        gather[b].wait()
        cur = bufs[b]

        def scale_row(r, carry):
            for v in range(VREGS_PER_ROW):
                sl = pl.ds(v * LANES, LANES)
                cur[r, sl] = cur[r, sl] * SCALE
            return carry

        lax.fori_loop(0, CHUNK, scale_row, 0, unroll=False)
        store[b] = pltpu.async_copy(
            cur, out_hbm.at[pl.ds(base + j * CHUNK, CHUNK)], ssems[b])
    for j in range(max(0, n_chunks - NBUF), n_chunks):
        store[j % NBUF].wait()


@functools.lru_cache(maxsize=None)
def _make_lookup(batch):
    assert batch % (NUM_WORKERS * CHUNK) == 0
    n_chunks = batch // (NUM_WORKERS * CHUNK)
    mesh = plsc.VectorSubcoreMesh(core_axis_name="c", subcore_axis_name="s")
    return pl.kernel(
        functools.partial(_sc_body, n_chunks),
        out_type=jax.ShapeDtypeStruct((batch, D_MODEL), jnp.float32),
        mesh=mesh,
        scratch_types=(
            [pltpu.VMEM((n_chunks, CHUNK), jnp.int32)]
            + [pltpu.VMEM((CHUNK, D_MODEL), jnp.float32)] * NBUF
            + [pltpu.SemaphoreType.DMA] * (2 * NBUF)
        ),
    )


def kernel(x, table):
    batch = x.size
    n_chunks = batch // (NUM_WORKERS * CHUNK)
    idx = x.astype(jnp.int32).reshape(NUM_WORKERS, n_chunks, CHUNK)
    out = _make_lookup(batch)(table, idx)
    return out.reshape(*x.shape, D_MODEL)


# speedup vs baseline: 1.1435x; 1.0898x over previous
import functools
import math

import jax
import jax.numpy as jnp
from jax import lax
from jax.experimental import pallas as pl
from jax.experimental.pallas import tpu as pltpu
from jax.experimental.pallas import tpu_sc as plsc

D_MODEL = 1024
SCALE = math.sqrt(D_MODEL)
LANES = 16
VREGS_PER_ROW = D_MODEL // LANES
NUM_CORES = 2
NUM_SUBCORES = 16
NUM_WORKERS = NUM_CORES * NUM_SUBCORES
CHUNK = 32
NBUF = 3


def _sc_body(n_chunks, table_hbm, idx_hbm, out_hbm, idx_v, *refs):
    bufs = refs[:NBUF]
    gsems = refs[NBUF:2 * NBUF]
    ssems = refs[2 * NBUF:3 * NBUF]
    wid = lax.axis_index("s") * NUM_CORES + lax.axis_index("c")
    base = wid * (n_chunks * CHUNK)
    pltpu.sync_copy(idx_hbm.at[wid], idx_v)

    gather = {}
    store = {}
    for j in range(min(NBUF - 1, n_chunks)):
        gather[j % NBUF] = pltpu.async_copy(
            table_hbm.at[idx_v.at[j]], bufs[j % NBUF], gsems[j % NBUF])
    for j in range(n_chunks):
        b = j % NBUF
        nxt = j + NBUF - 1
        if nxt < n_chunks:
            nb = nxt % NBUF
            if nxt >= NBUF:
                store[nb].wait()
            gather[nb] = pltpu.async_copy(
                table_hbm.at[idx_v.at[nxt]], bufs[nb], gsems[nb])
        gather[b].wait()
        cur = bufs[b]

        @plsc.parallel_loop(0, CHUNK, 1)
        def scale_row(r):
            for v in range(VREGS_PER_ROW):
                sl = pl.ds(v * LANES, LANES)
                cur[r, sl] = cur[r, sl] * SCALE
        store[b] = pltpu.async_copy(
            cur, out_hbm.at[pl.ds(base + j * CHUNK, CHUNK)], ssems[b])
    for j in range(max(0, n_chunks - NBUF), n_chunks):
        store[j % NBUF].wait()


@functools.lru_cache(maxsize=None)
def _make_lookup(batch):
    assert batch % (NUM_WORKERS * CHUNK) == 0
    n_chunks = batch // (NUM_WORKERS * CHUNK)
    mesh = plsc.VectorSubcoreMesh(core_axis_name="c", subcore_axis_name="s")
    return pl.kernel(
        functools.partial(_sc_body, n_chunks),
        out_type=jax.ShapeDtypeStruct((batch, D_MODEL), jnp.float32),
        mesh=mesh,
        scratch_types=(
            [pltpu.VMEM((n_chunks, CHUNK), jnp.int32)]
            + [pltpu.VMEM((CHUNK, D_MODEL), jnp.float32)] * NBUF
            + [pltpu.SemaphoreType.DMA] * (2 * NBUF)
        ),
    )


def kernel(x, table):
    batch = x.size
    n_chunks = batch // (NUM_WORKERS * CHUNK)
    idx = x.astype(jnp.int32).reshape(NUM_WORKERS, n_chunks, CHUNK)
    out = _make_lookup(batch)(table, idx)
    return out.reshape(*x.shape, D_MODEL)
